# Initial kernel scaffold; baseline (speedup 1.0000x reference)
import jax
import jax.numpy as jnp
from jax.experimental import pallas as pl


def kernel(logits, mask):
    raise NotImplementedError("write your pallas kernel here")


# speedup vs baseline: 1.0433x; 1.0433x over previous
import jax
import jax.numpy as jnp
from jax.experimental import pallas as pl

_NEG = -1e9
_TINY = 1.1754943508222875e-38
_KS0 = 0x0
_KS1 = 0x2A
_KS2 = _KS0 ^ _KS1 ^ 0x1BD11BDA
_ROT_A = (13, 15, 26, 6)
_ROT_B = (17, 29, 16, 24)


def _rotl(x, d):
    return (x << jnp.uint32(d)) | (x >> jnp.uint32(32 - d))


def _threefry_rounds(x0, x1, rots):
    for r in rots:
        x0 = x0 + x1
        x1 = _rotl(x1, r)
        x1 = x0 ^ x1
    return x0, x1


def _threefry_bits(cnt_hi, cnt_lo):
    ks0 = jnp.uint32(_KS0)
    ks1 = jnp.uint32(_KS1)
    ks2 = jnp.uint32(_KS2)
    x0 = cnt_hi + ks0
    x1 = cnt_lo + ks1
    x0, x1 = _threefry_rounds(x0, x1, _ROT_A)
    x0, x1 = x0 + ks1, x1 + ks2 + jnp.uint32(1)
    x0, x1 = _threefry_rounds(x0, x1, _ROT_B)
    x0, x1 = x0 + ks2, x1 + ks0 + jnp.uint32(2)
    x0, x1 = _threefry_rounds(x0, x1, _ROT_A)
    x0, x1 = x0 + ks0, x1 + ks1 + jnp.uint32(3)
    x0, x1 = _threefry_rounds(x0, x1, _ROT_B)
    x0, x1 = x0 + ks1, x1 + ks2 + jnp.uint32(4)
    x0, x1 = _threefry_rounds(x0, x1, _ROT_A)
    x0, x1 = x0 + ks2, x1 + ks0 + jnp.uint32(5)
    return x0 ^ x1


def _body(logits_ref, mask_ref, out_ref, *, vshift):
    blk_r, v = logits_ref.shape
    logits = logits_ref[...]
    mask = mask_ref[...]
    masked = jnp.where(mask, logits, jnp.float32(_NEG))

    m = jnp.max(masked, axis=1, keepdims=True)
    s = jnp.sum(jnp.exp(masked - m), axis=1, keepdims=True)
    lse = m + jnp.log(s)

    row0 = jnp.uint32(pl.program_id(0) * blk_r)
    rowi = jax.lax.broadcasted_iota(jnp.uint32, (blk_r, v), 0)
    coli = jax.lax.broadcasted_iota(jnp.uint32, (blk_r, v), 1)
    cnt_lo = ((row0 + rowi) << jnp.uint32(vshift)) | coli
    bits = _threefry_bits(jnp.zeros((blk_r, v), jnp.uint32), cnt_lo)

    fb = (bits >> jnp.uint32(9)) | jnp.uint32(0x3F800000)
    f = jax.lax.bitcast_convert_type(fb, jnp.float32) - jnp.float32(1.0)
    tiny = jnp.float32(_TINY)
    u = jnp.maximum(tiny, f * (jnp.float32(1.0) - tiny) + tiny)
    g = -jnp.log(-jnp.log(u))

    y = g + masked
    ymax = jnp.max(y, axis=1, keepdims=True)
    ci32 = coli.astype(jnp.int32)
    amax = jnp.min(jnp.where(y == ymax, ci32, jnp.int32(v)), axis=1,
                   keepdims=True)
    val = jnp.sum(jnp.where(ci32 == amax, masked, jnp.float32(0.0)), axis=1,
                  keepdims=True)
    out_ref[...] = val - lse


def kernel(logits, mask):
    b, v = logits.shape
    assert (v & (v - 1)) == 0, "V must be a power of two"
    vshift = v.bit_length() - 1
    blk_r = 128 if b % 128 == 0 else b

    import functools
    out = pl.pallas_call(
        functools.partial(_body, vshift=vshift),
        grid=(b // blk_r,),
        in_specs=[
            pl.BlockSpec((blk_r, v), lambda i: (i, 0)),
            pl.BlockSpec((blk_r, v), lambda i: (i, 0)),
        ],
        out_specs=pl.BlockSpec((blk_r, 1), lambda i: (i, 0)),
        out_shape=jax.ShapeDtypeStruct((b, 1), jnp.float32),
    )(logits, mask)
    return out.reshape(b)


# speedup vs baseline: 1.0512x; 1.0075x over previous
import jax
import jax.numpy as jnp
from jax.experimental import pallas as pl

_NEG = -1e9
_TINY = 1.1754943508222875e-38
_KS0 = 0x0
_KS1 = 0x2A
_KS2 = _KS0 ^ _KS1 ^ 0x1BD11BDA
_ROT_A = (13, 15, 26, 6)
_ROT_B = (17, 29, 16, 24)


def _rotl(x, d):
    return (x << jnp.uint32(d)) | (x >> jnp.uint32(32 - d))


def _threefry_rounds(x0, x1, rots):
    for r in rots:
        x0 = x0 + x1
        x1 = _rotl(x1, r)
        x1 = x0 ^ x1
    return x0, x1


def _threefry_bits(cnt_hi, cnt_lo):
    ks0 = jnp.uint32(_KS0)
    ks1 = jnp.uint32(_KS1)
    ks2 = jnp.uint32(_KS2)
    x0 = cnt_hi + ks0
    x1 = cnt_lo + ks1
    x0, x1 = _threefry_rounds(x0, x1, _ROT_A)
    x0, x1 = x0 + ks1, x1 + ks2 + jnp.uint32(1)
    x0, x1 = _threefry_rounds(x0, x1, _ROT_B)
    x0, x1 = x0 + ks2, x1 + ks0 + jnp.uint32(2)
    x0, x1 = _threefry_rounds(x0, x1, _ROT_A)
    x0, x1 = x0 + ks0, x1 + ks1 + jnp.uint32(3)
    x0, x1 = _threefry_rounds(x0, x1, _ROT_B)
    x0, x1 = x0 + ks1, x1 + ks2 + jnp.uint32(4)
    x0, x1 = _threefry_rounds(x0, x1, _ROT_A)
    x0, x1 = x0 + ks2, x1 + ks0 + jnp.uint32(5)
    return x0 ^ x1


def _body(logits_ref, mask_ref, out_ref, *, vshift):
    blk_r, v = logits_ref.shape
    logits = logits_ref[...]
    mask = mask_ref[...]
    masked = jnp.where(mask, logits, jnp.float32(_NEG))

    m = jnp.max(masked, axis=1, keepdims=True)
    s = jnp.sum(jnp.exp(masked - m), axis=1, keepdims=True)
    lse = m + jnp.log(s)

    row0 = jnp.uint32(pl.program_id(0) * blk_r)
    rowi = jax.lax.broadcasted_iota(jnp.uint32, (blk_r, v), 0)
    coli = jax.lax.broadcasted_iota(jnp.uint32, (blk_r, v), 1)
    cnt_lo = ((row0 + rowi) << jnp.uint32(vshift)) | coli
    bits = _threefry_bits(jnp.zeros((blk_r, v), jnp.uint32), cnt_lo)

    fb = (bits >> jnp.uint32(9)) | jnp.uint32(0x3F800000)
    f = jax.lax.bitcast_convert_type(fb, jnp.float32) - jnp.float32(1.0)
    tiny = jnp.float32(_TINY)
    u = jnp.maximum(tiny, f * (jnp.float32(1.0) - tiny) + tiny)
    g = -jnp.log(-jnp.log(u))

    y = g + masked
    ymax = jnp.max(y, axis=1, keepdims=True)
    ci32 = coli.astype(jnp.int32)
    amax = jnp.min(jnp.where(y == ymax, ci32, jnp.int32(v)), axis=1,
                   keepdims=True)
    val = jnp.sum(jnp.where(ci32 == amax, masked, jnp.float32(0.0)), axis=1,
                  keepdims=True)
    out_ref[...] = val - lse


def kernel(logits, mask):
    b, v = logits.shape
    assert (v & (v - 1)) == 0, "V must be a power of two"
    vshift = v.bit_length() - 1
    blk_r = 256 if b % 256 == 0 else b

    import functools
    out = pl.pallas_call(
        functools.partial(_body, vshift=vshift),
        grid=(b // blk_r,),
        in_specs=[
            pl.BlockSpec((blk_r, v), lambda i: (i, 0)),
            pl.BlockSpec((blk_r, v), lambda i: (i, 0)),
        ],
        out_specs=pl.BlockSpec((blk_r, 1), lambda i: (i, 0)),
        out_shape=jax.ShapeDtypeStruct((b, 1), jnp.float32),
    )(logits, mask)
    return out.reshape(b)
